# Initial kernel scaffold; baseline (speedup 1.0000x reference)
import jax
import jax.numpy as jnp
from jax.experimental import pallas as pl


def kernel(x, edge_index, W1, b1, W2, b2):
    raise NotImplementedError("write your pallas kernel here")


# speedup vs baseline: 6.9348x; 6.9348x over previous
import functools

import jax
import jax.numpy as jnp
from jax import lax
from jax.experimental import pallas as pl
from jax.experimental.pallas import tpu as pltpu
from jax.experimental.pallas import tpu_sc as plsc

N = 10000
D = 128
D_HID = 256
NPAD = 10240
NC = 2
NS = 16
NW = NC * NS
CHUNK = 128
BM = 512


def _make_spmm(e_pad, with_cnt):
  per_worker = e_pad // NW
  n_chunks = per_worker // CHUNK
  rps = NPAD // NS

  mesh = plsc.VectorSubcoreMesh(core_axis_name="c", subcore_axis_name="s")
  out_type = [jax.ShapeDtypeStruct((NC, NPAD, D), jnp.float32)]
  scratch = [
      pltpu.VMEM_SHARED((NPAD, D), jnp.float32),
      pltpu.VMEM((CHUNK,), jnp.int32),
      pltpu.VMEM((CHUNK,), jnp.int32),
      pltpu.VMEM((CHUNK, D), jnp.float32),
      pltpu.SemaphoreType.DMA,
  ]
  if with_cnt:
    out_type.append(jax.ShapeDtypeStruct((NC, NPAD), jnp.float32))
    scratch.append(pltpu.VMEM_SHARED((NPAD,), jnp.float32))
    scratch.append(pltpu.VMEM((CHUNK,), jnp.float32))

  def body(*refs):
    if with_cnt:
      (h_hbm, row_hbm, col_hbm, zr_hbm, zc_hbm, p_hbm, cnt_hbm,
       acc, ridx, cidx, rows, sem, cacc, ones) = refs
    else:
      (h_hbm, row_hbm, col_hbm, zr_hbm, p_hbm,
       acc, ridx, cidx, rows, sem) = refs
    c = lax.axis_index("c")
    s = lax.axis_index("s")
    wid = c * NS + s
    rbase = pl.multiple_of(s * rps, 8)

    pltpu.sync_copy(zr_hbm.at[pl.ds(rbase, rps)], acc.at[pl.ds(rbase, rps)])
    if with_cnt:
      pltpu.sync_copy(zc_hbm.at[pl.ds(rbase, rps)], cacc.at[pl.ds(rbase, rps)])

      @pl.loop(0, CHUNK, step=16)
      def _(k):
        ones[pl.ds(k, 16)] = jnp.ones((16,), jnp.float32)

    plsc.subcore_barrier()

    base = wid * per_worker

    @pl.loop(0, n_chunks)
    def _(i):
      off = pl.multiple_of(base + i * CHUNK, 8)
      pltpu.sync_copy(row_hbm.at[pl.ds(off, CHUNK)], ridx)
      pltpu.sync_copy(col_hbm.at[pl.ds(off, CHUNK)], cidx)
      pltpu.async_copy(h_hbm.at[ridx], rows, sem).wait()
      pltpu.sync_copy(rows, acc.at[cidx], add=True)
      if with_cnt:
        pltpu.sync_copy(ones, cacc.at[cidx], add=True)

    plsc.subcore_barrier()

    pltpu.sync_copy(acc.at[pl.ds(rbase, rps)],
                    p_hbm.at[c].at[pl.ds(rbase, rps)])
    if with_cnt:
      pltpu.sync_copy(cacc.at[pl.ds(rbase, rps)],
                      cnt_hbm.at[c].at[pl.ds(rbase, rps)])

  return pl.kernel(body, out_type=tuple(out_type), mesh=mesh,
                   scratch_types=scratch)


def _dense_body(p0, p1, c0, c1, w1t, b1r, w2t, out):
  s = p0[...] + p1[...]
  cc = c0[...] + c1[...]
  inv = jnp.where(cc > 0.0, 1.0 / cc, 0.0)
  h = jnp.dot(s * inv, w1t[...], preferred_element_type=jnp.float32)
  h = jnp.maximum(h + b1r[...], 0.0)
  out[...] = jnp.dot(h, w2t[...], preferred_element_type=jnp.float32)


def _finish_body(p0, p1, c0, c1, b2r, out):
  s = p0[...] + p1[...]
  cc = c0[...] + c1[...]
  inv = jnp.where(cc > 0.0, 1.0 / cc, 0.0)
  r = s * inv + b2r[...]
  nrm = jnp.sqrt(jnp.sum(r * r, axis=1, keepdims=True))
  out[...] = r / jnp.maximum(nrm, 1e-12)


def _row_specs():
  return [
      pl.BlockSpec((BM, D), lambda i: (i, 0)),
      pl.BlockSpec((BM, D), lambda i: (i, 0)),
      pl.BlockSpec((BM, 1), lambda i: (i, 0)),
      pl.BlockSpec((BM, 1), lambda i: (i, 0)),
  ]


@jax.jit
def kernel(x, edge_index, W1, b1, W2, b2):
  e = edge_index.shape[1]
  e_pad = -(-e // (NW * CHUNK)) * (NW * CHUNK)
  row = edge_index[0]
  col = edge_index[1]
  if e_pad != e:
    row = jnp.concatenate([row, jnp.zeros((e_pad - e,), jnp.int32)])
    col = jnp.concatenate([col, jnp.full((e_pad - e,), N, jnp.int32)])
  zr = jnp.zeros((NPAD, D), jnp.float32)
  zc = jnp.zeros((NPAD,), jnp.float32)

  p1, cnt = _make_spmm(e_pad, True)(x, row, col, zr, zc)
  c0 = cnt[0][:, None]
  c1 = cnt[1][:, None]

  grid = (NPAD // BM,)
  b = pl.pallas_call(
      _dense_body,
      grid=grid,
      in_specs=_row_specs() + [
          pl.BlockSpec((D, D_HID), lambda i: (0, 0)),
          pl.BlockSpec((1, D_HID), lambda i: (0, 0)),
          pl.BlockSpec((D_HID, D), lambda i: (0, 0)),
      ],
      out_specs=pl.BlockSpec((BM, D), lambda i: (i, 0)),
      out_shape=jax.ShapeDtypeStruct((NPAD, D), jnp.float32),
  )(p1[0], p1[1], c0, c1, W1.T, b1[None, :], W2.T)

  (p2,) = _make_spmm(e_pad, False)(b, row, col, zr)

  out = pl.pallas_call(
      _finish_body,
      grid=grid,
      in_specs=_row_specs() + [pl.BlockSpec((1, D), lambda i: (0, 0))],
      out_specs=pl.BlockSpec((BM, D), lambda i: (i, 0)),
      out_shape=jax.ShapeDtypeStruct((NPAD, D), jnp.float32),
  )(p2[0], p2[1], c0, c1, b2[None, :])
  return out[:N]
